# baseline (speedup 1.0000x reference)
import jax
import jax.numpy as jnp
from jax.experimental import pallas as pl

_BANK = 64
_BS0 = 16


def _combine_kernel(probs_ref, idx_ref, param_ref, out_ref):
    probs = probs_ref[...]
    idx = idx_ref[...]
    b, k = probs.shape
    lanes = jax.lax.broadcasted_iota(jnp.int32, (b, _BANK), 1)
    w = jnp.zeros((b, _BANK), jnp.float32)
    for kk in range(k):
        w = w + jnp.where(idx[:, kk:kk + 1] == lanes, probs[:, kk:kk + 1], 0.0)
    v = param_ref[...]
    for i in range(v.shape[0]):
        out_ref[:, i, :] = jax.lax.dot_general(
            w, v[i], (((1,), (0,)), ((), ())),
            preferred_element_type=jnp.float32)


def kernel(selection_probabilities, parameter, selection_index):
    s0, s1, bank = parameter.shape
    b, k = selection_index.shape
    pview = jnp.transpose(parameter, (0, 2, 1))
    grid = s0 // _BS0
    out = pl.pallas_call(
        _combine_kernel,
        grid=(grid,),
        in_specs=[
            pl.BlockSpec((b, k), lambda i: (0, 0)),
            pl.BlockSpec((b, k), lambda i: (0, 0)),
            pl.BlockSpec((_BS0, bank, s1), lambda i: (i, 0, 0)),
        ],
        out_specs=pl.BlockSpec((b, _BS0, s1), lambda i: (0, i, 0)),
        out_shape=jax.ShapeDtypeStruct((b, s0, s1), jnp.float32),
    )(selection_probabilities, selection_index, pview)
    return out


# speedup vs baseline: 1.1141x; 1.1141x over previous
import jax
import jax.numpy as jnp
from jax.experimental import pallas as pl

_BANK = 64
_BS0 = 32


def _combine_kernel(probs_ref, idx_ref, param_ref, out_ref):
    probs = probs_ref[...]
    idx = idx_ref[...]
    b, k = probs.shape
    lanes = jax.lax.broadcasted_iota(jnp.int32, (b, _BANK), 1)
    w = jnp.zeros((b, _BANK), jnp.float32)
    for kk in range(k):
        w = w + jnp.where(idx[:, kk:kk + 1] == lanes, probs[:, kk:kk + 1], 0.0)
    v = param_ref[...]
    for i in range(v.shape[0]):
        out_ref[:, i, :] = jax.lax.dot_general(
            w, v[i], (((1,), (0,)), ((), ())),
            preferred_element_type=jnp.float32)


def kernel(selection_probabilities, parameter, selection_index):
    s0, s1, bank = parameter.shape
    b, k = selection_index.shape
    pview = jnp.transpose(parameter, (0, 2, 1))
    grid = s0 // _BS0
    out = pl.pallas_call(
        _combine_kernel,
        grid=(grid,),
        in_specs=[
            pl.BlockSpec((b, k), lambda i: (0, 0)),
            pl.BlockSpec((b, k), lambda i: (0, 0)),
            pl.BlockSpec((_BS0, bank, s1), lambda i: (i, 0, 0)),
        ],
        out_specs=pl.BlockSpec((b, _BS0, s1), lambda i: (0, i, 0)),
        out_shape=jax.ShapeDtypeStruct((b, s0, s1), jnp.float32),
    )(selection_probabilities, selection_index, pview)
    return out


# speedup vs baseline: 1.2283x; 1.1025x over previous
import jax
import jax.numpy as jnp
from jax.experimental import pallas as pl
from jax.experimental.pallas import tpu as pltpu

_BS0 = 512
_SUB = 8
_LANE = 128


def _make_kernel(total, bs0):
    def body(idx_sref, probs_sref, pv_ref, out_ref, buf_ref, sem_ref):
        i = pl.program_id(0)
        b = pl.program_id(1)
        k2 = pl.program_id(2)
        nb = pl.num_programs(1)
        nk = pl.num_programs(2)
        flat = (i * nb + b) * nk + k2

        def copy_for(f, slot):
            i_f = f // (nb * nk)
            r = f % (nb * nk)
            c = idx_sref[r]
            src = pv_ref.at[pl.ds(i_f * bs0, bs0), c // _SUB, :, c % _SUB, :]
            return pltpu.make_async_copy(src, buf_ref.at[slot],
                                         sem_ref.at[slot])

        @pl.when(flat == 0)
        def _first():
            copy_for(flat, flat % 2).start()

        @pl.when(flat + 1 < total)
        def _prefetch():
            copy_for(flat + 1, (flat + 1) % 2).start()

        copy_for(flat, flat % 2).wait()
        v = buf_ref[flat % 2]
        p = probs_sref[b * nk + k2]

        @pl.when(k2 == 0)
        def _init():
            out_ref[0] = p * v

        @pl.when(k2 != 0)
        def _acc():
            out_ref[0] += p * v

    return body


def kernel(selection_probabilities, parameter, selection_index):
    s0, s1, bank = parameter.shape
    b, k = selection_index.shape
    cb, jb = bank // _SUB, s1 // _LANE
    pv = jnp.transpose(parameter, (0, 2, 1))
    pv = pv.reshape(s0, cb, _SUB, jb, _LANE)
    pv = jnp.transpose(pv, (0, 1, 3, 2, 4))
    idx_flat = selection_index.reshape(-1)
    probs_flat = selection_probabilities.reshape(-1)
    grid = (s0 // _BS0, b, k)
    total = (s0 // _BS0) * b * k
    out = pl.pallas_call(
        _make_kernel(total, _BS0),
        grid_spec=pltpu.PrefetchScalarGridSpec(
            num_scalar_prefetch=2,
            grid=grid,
            in_specs=[pl.BlockSpec(memory_space=pl.ANY)],
            out_specs=pl.BlockSpec(
                (1, _BS0, jb, _LANE),
                lambda i, bb, kk, idx, pr: (bb, i, 0, 0)),
            scratch_shapes=[
                pltpu.VMEM((2, _BS0, jb, _LANE), jnp.float32),
                pltpu.SemaphoreType.DMA((2,)),
            ],
        ),
        out_shape=jax.ShapeDtypeStruct((b, s0, jb, _LANE), jnp.float32),
    )(idx_flat, probs_flat, pv)
    return out.reshape(b, s0, s1)


# speedup vs baseline: 1.8646x; 1.5180x over previous
import jax
import jax.numpy as jnp
from jax.experimental import pallas as pl
from jax.experimental.pallas import tpu as pltpu

_BS0 = 512
_SUB = 8
_LANE = 128


def _make_kernel(total, bs0):
    def body(idx_sref, probs_sref, pv_ref, out_ref, buf_ref, sem_ref):
        i = pl.program_id(0)
        b = pl.program_id(1)
        k2 = pl.program_id(2)
        nb = pl.num_programs(1)
        nk = pl.num_programs(2)
        flat = (i * nb + b) * nk + k2

        def copy_for(f, slot):
            i_f = f // (nb * nk)
            r = f % (nb * nk)
            c = idx_sref[r]
            src = pv_ref.at[pl.ds(i_f * bs0, bs0), c // _SUB, :, c % _SUB, :]
            return pltpu.make_async_copy(src, buf_ref.at[slot],
                                         sem_ref.at[slot])

        @pl.when(flat == 0)
        def _first():
            copy_for(flat, flat % 2).start()

        @pl.when(flat + 1 < total)
        def _prefetch():
            copy_for(flat + 1, (flat + 1) % 2).start()

        copy_for(flat, flat % 2).wait()
        v = buf_ref[flat % 2]
        p = probs_sref[b * nk + k2]

        pv2 = (p * v).reshape(v.shape[0], v.shape[1] * v.shape[2])

        @pl.when(k2 == 0)
        def _init():
            out_ref[0] = pv2

        @pl.when(k2 != 0)
        def _acc():
            out_ref[0] += pv2

    return body


def kernel(selection_probabilities, parameter, selection_index):
    s0, s1, bank = parameter.shape
    b, k = selection_index.shape
    cb, jb = bank // _SUB, s1 // _LANE
    pv = jnp.transpose(parameter, (0, 2, 1))
    pv = pv.reshape(s0, cb, _SUB, jb, _LANE)
    pv = jnp.transpose(pv, (0, 1, 3, 2, 4))
    idx_flat = selection_index.reshape(-1)
    probs_flat = selection_probabilities.reshape(-1)
    grid = (s0 // _BS0, b, k)
    total = (s0 // _BS0) * b * k
    out = pl.pallas_call(
        _make_kernel(total, _BS0),
        grid_spec=pltpu.PrefetchScalarGridSpec(
            num_scalar_prefetch=2,
            grid=grid,
            in_specs=[pl.BlockSpec(memory_space=pl.ANY)],
            out_specs=pl.BlockSpec(
                (1, _BS0, s1),
                lambda i, bb, kk, idx, pr: (bb, i, 0)),
            scratch_shapes=[
                pltpu.VMEM((2, _BS0, jb, _LANE), jnp.float32),
                pltpu.SemaphoreType.DMA((2,)),
            ],
        ),
        out_shape=jax.ShapeDtypeStruct((b, s0, s1), jnp.float32),
    )(idx_flat, probs_flat, pv)
    return out


# speedup vs baseline: 2.1061x; 1.1296x over previous
import jax
import jax.numpy as jnp
from jax.experimental import pallas as pl
from jax.experimental.pallas import tpu as pltpu

_BS0 = 256
_SUB = 8
_LANE = 128


def _make_kernel(total, bs0):
    def body(idx_sref, probs_sref, pv_ref, out_ref, buf_ref, sem_ref):
        i = pl.program_id(0)
        b = pl.program_id(1)
        k2 = pl.program_id(2)
        nb = pl.num_programs(1)
        nk = pl.num_programs(2)
        flat = (i * nb + b) * nk + k2

        def copy_for(f, slot):
            i_f = f // (nb * nk)
            r = f % (nb * nk)
            c = idx_sref[r]
            src = pv_ref.at[pl.ds(i_f * bs0, bs0), c // _SUB, :, c % _SUB, :]
            return pltpu.make_async_copy(src, buf_ref.at[slot],
                                         sem_ref.at[slot])

        @pl.when(flat == 0)
        def _first():
            for f in range(min(7, total)):
                copy_for(f, f % 8).start()

        @pl.when(flat + 7 < total)
        def _prefetch():
            copy_for(flat + 7, (flat + 7) % 8).start()

        copy_for(flat, flat % 8).wait()
        v = buf_ref[flat % 8]
        p = probs_sref[b * nk + k2]

        pv2 = (p * v).reshape(v.shape[0], v.shape[1] * v.shape[2])

        @pl.when(k2 == 0)
        def _init():
            out_ref[0] = pv2

        @pl.when(k2 != 0)
        def _acc():
            out_ref[0] += pv2

    return body


def kernel(selection_probabilities, parameter, selection_index):
    s0, s1, bank = parameter.shape
    b, k = selection_index.shape
    cb, jb = bank // _SUB, s1 // _LANE
    pv = jnp.transpose(parameter, (0, 2, 1))
    pv = pv.reshape(s0, cb, _SUB, jb, _LANE)
    pv = jnp.transpose(pv, (0, 1, 3, 2, 4))
    idx_flat = selection_index.reshape(-1)
    probs_flat = selection_probabilities.reshape(-1)
    grid = (s0 // _BS0, b, k)
    total = (s0 // _BS0) * b * k
    out = pl.pallas_call(
        _make_kernel(total, _BS0),
        grid_spec=pltpu.PrefetchScalarGridSpec(
            num_scalar_prefetch=2,
            grid=grid,
            in_specs=[pl.BlockSpec(memory_space=pl.ANY)],
            out_specs=pl.BlockSpec(
                (1, _BS0, s1),
                lambda i, bb, kk, idx, pr: (bb, i, 0)),
            scratch_shapes=[
                pltpu.VMEM((8, _BS0, jb, _LANE), jnp.float32),
                pltpu.SemaphoreType.DMA((8,)),
            ],
        ),
        out_shape=jax.ShapeDtypeStruct((b, s0, s1), jnp.float32),
    )(idx_flat, probs_flat, pv)
    return out


# speedup vs baseline: 2.5578x; 1.2144x over previous
import jax
import jax.numpy as jnp
from jax.experimental import pallas as pl
from jax.experimental.pallas import tpu as pltpu

_BS0 = 512
_SUB = 8
_LANE = 128


def _make_kernel(total, bs0):
    def body(idx_sref, probs_sref, pv_ref, out_ref, buf_ref, sem_ref):
        i = pl.program_id(0)
        b = pl.program_id(1)
        k2 = pl.program_id(2)
        nb = pl.num_programs(1)
        nk = pl.num_programs(2)
        flat = (i * nb + b) * nk + k2

        def copy_for(f, slot):
            i_f = f // (nb * nk)
            r = f % (nb * nk)
            c = idx_sref[r]
            src = pv_ref.at[pl.ds(i_f * bs0, bs0), c // _SUB, :, c % _SUB, :]
            return pltpu.make_async_copy(src, buf_ref.at[slot],
                                         sem_ref.at[slot])

        @pl.when(flat == 0)
        def _first():
            for f in range(min(7, total)):
                copy_for(f, f % 8).start()

        @pl.when(flat + 7 < total)
        def _prefetch():
            copy_for(flat + 7, (flat + 7) % 8).start()

        copy_for(flat, flat % 8).wait()
        v = buf_ref[flat % 8]
        p = probs_sref[b * nk + k2]

        pv2 = (p * v).reshape(v.shape[0], v.shape[1] * v.shape[2])

        @pl.when(k2 == 0)
        def _init():
            out_ref[0] = pv2

        @pl.when(k2 != 0)
        def _acc():
            out_ref[0] += pv2

    return body


def kernel(selection_probabilities, parameter, selection_index):
    s0, s1, bank = parameter.shape
    b, k = selection_index.shape
    cb, jb = bank // _SUB, s1 // _LANE
    pv = jnp.transpose(parameter, (0, 2, 1))
    pv = pv.reshape(s0, cb, _SUB, jb, _LANE)
    pv = jnp.transpose(pv, (0, 1, 3, 2, 4))
    idx_flat = selection_index.reshape(-1)
    probs_flat = selection_probabilities.reshape(-1)
    grid = (s0 // _BS0, b, k)
    total = (s0 // _BS0) * b * k
    out = pl.pallas_call(
        _make_kernel(total, _BS0),
        grid_spec=pltpu.PrefetchScalarGridSpec(
            num_scalar_prefetch=2,
            grid=grid,
            in_specs=[pl.BlockSpec(memory_space=pl.ANY)],
            out_specs=pl.BlockSpec(
                (1, _BS0, s1),
                lambda i, bb, kk, idx, pr: (bb, i, 0)),
            scratch_shapes=[
                pltpu.VMEM((8, _BS0, jb, _LANE), jnp.float32),
                pltpu.SemaphoreType.DMA((8,)),
            ],
        ),
        out_shape=jax.ShapeDtypeStruct((b, s0, s1), jnp.float32),
    )(idx_flat, probs_flat, pv)
    return out


# speedup vs baseline: 2.9201x; 1.1417x over previous
import jax
import jax.numpy as jnp
from jax.experimental import pallas as pl
from jax.experimental.pallas import tpu as pltpu

_BS0 = 512
_SLOTS = 8
_SUB = 8
_LANE = 128


def _make_kernel(total_fetches, bs0, nk, s1):
    lookahead = _SLOTS // nk - 1

    def body(idx_sref, probs_sref, pv_ref, out_ref, buf_ref, sem_ref):
        i = pl.program_id(0)
        b = pl.program_id(1)
        nb = pl.num_programs(1)
        flat = i * nb + b

        def copy_for(g, slot):
            i_g = g // (nb * nk)
            c = idx_sref[g % (nb * nk)]
            src = pv_ref.at[pl.ds(i_g * bs0, bs0), c // _SUB, :, c % _SUB, :]
            return pltpu.make_async_copy(src, buf_ref.at[slot],
                                         sem_ref.at[slot])

        @pl.when(flat == 0)
        def _first():
            for g in range(min(lookahead * nk, total_fetches)):
                copy_for(g, g % _SLOTS).start()

        g_next = (flat + lookahead) * nk
        for kk in range(nk):
            @pl.when(g_next + kk < total_fetches)
            def _prefetch(kk=kk):
                copy_for(g_next + kk, (g_next + kk) % _SLOTS).start()

        acc = None
        for kk in range(nk):
            g = flat * nk + kk
            copy_for(g, g % _SLOTS).wait()
            v = buf_ref[g % _SLOTS]
            p = probs_sref[b * nk + kk]
            acc = p * v if acc is None else acc + p * v
        out_ref[0] = acc.reshape(bs0, s1)

    return body


def kernel(selection_probabilities, parameter, selection_index):
    s0, s1, bank = parameter.shape
    b, k = selection_index.shape
    cb, jb = bank // _SUB, s1 // _LANE
    pv = jnp.transpose(parameter, (0, 2, 1))
    pv = pv.reshape(s0, cb, _SUB, jb, _LANE)
    pv = jnp.transpose(pv, (0, 1, 3, 2, 4))
    idx_flat = selection_index.reshape(-1)
    probs_flat = selection_probabilities.reshape(-1)
    grid = (s0 // _BS0, b)
    total_fetches = (s0 // _BS0) * b * k
    out = pl.pallas_call(
        _make_kernel(total_fetches, _BS0, k, s1),
        grid_spec=pltpu.PrefetchScalarGridSpec(
            num_scalar_prefetch=2,
            grid=grid,
            in_specs=[pl.BlockSpec(memory_space=pl.ANY)],
            out_specs=pl.BlockSpec(
                (1, _BS0, s1),
                lambda i, bb, idx, pr: (bb, i, 0)),
            scratch_shapes=[
                pltpu.VMEM((_SLOTS, _BS0, jb, _LANE), jnp.float32),
                pltpu.SemaphoreType.DMA((_SLOTS,)),
            ],
        ),
        out_shape=jax.ShapeDtypeStruct((b, s0, s1), jnp.float32),
    )(idx_flat, probs_flat, pv)
    return out
